# baseline (speedup 1.0000x reference)
import jax
import jax.numpy as jnp
from jax.experimental import pallas as pl
from jax.experimental.pallas import tpu as pltpu

_F32 = jnp.float32
_PAD = 128
_TT = 512
_TLAT = 2048


def _lrelu(v):
    return jnp.where(v >= 0, v, 0.01 * v)


def _dot(a, b):
    return jnp.dot(a, b, preferred_element_type=_F32)


def _zero_guards(ref, t=_TLAT):
    c = ref.shape[0]
    ref[:, _PAD - 1:_PAD] = jnp.zeros((c, 1), _F32)
    ref[:, _PAD + t:_PAD + t + 1] = jnp.zeros((c, 1), _F32)


def _wblock(rows, kg, placements):
    w = jnp.zeros((rows, kg), _F32)
    for (r0, k0, m) in placements:
        w = w.at[r0:r0 + m.shape[0], k0:k0 + m.shape[1]].set(m)
    return w


def _run_layer(in_ref, out_ref, blocks, act, out_off):
    def tile(j, _):
        t0 = j * _TT
        for (out0, bias, groups) in blocks:
            acc = None
            for (wv, in0, sh) in groups:
                kg = wv.shape[1]
                sa = in_ref[in0:in0 + kg, pl.ds(t0, _TT + 2 * _PAD)]
                seg = sa[:, _PAD + sh:_PAD + sh + _TT]
                d = _dot(wv, seg)
                acc = d if acc is None else acc + d
            v = acc + bias
            if act:
                v = _lrelu(v)
            nr = v.shape[0]
            out_ref[out0:out0 + nr, pl.ds(t0 + out_off, _TT)] = v
        return 0

    jax.lax.fori_loop(0, _TLAT // _TT, tile, 0)


def _col(v):
    return v.reshape(-1, 1)


def _build_enc_layers(ec1_w, ec1_b, ec2_w, ec2_b, ec3_w, ec3_b, ec4_w, ec4_b):
    w1 = jnp.transpose(ec1_w, (2, 0, 1))
    w2 = jnp.transpose(ec2_w, (2, 0, 1))
    w3 = jnp.transpose(ec3_w, (2, 0, 1))
    w4 = jnp.transpose(ec4_w, (2, 0, 1))

    sh0 = []
    for q in range(4):
        for dk in range(4):
            s = 2 * q + dk - 1
            if 0 <= s <= 7:
                sh0.append((q * 128, s * 2, w1[dk]))
    l1 = dict(act=True, blocks=[
        (0, jnp.tile(_col(ec1_b), (4, 1)), [
            (_wblock(512, 16, sh0), 0, 0),
            (_wblock(512, 2, [(0, 0, w1[0])]), 14, -1),
            (_wblock(512, 2, [(384, 0, w1[3])]), 0, 1),
        ])])

    b2 = _col(ec2_b)
    l2 = dict(act=True, blocks=[
        (0, b2, [
            (jnp.concatenate([w2[1], w2[2], w2[3]], axis=1), 0, 0),
            (w2[0], 384, -1),
        ]),
        (256, b2, [
            (jnp.concatenate([w2[0], w2[1], w2[2]], axis=1), 128, 0),
            (w2[3], 0, 1),
        ])])

    l3 = dict(act=True, blocks=[
        (0, _col(ec3_b), [
            (jnp.concatenate([w3[1], w3[2]], axis=1), 0, 0),
            (w3[0], 256, -1),
            (w3[3], 0, 1),
        ])])

    l4 = dict(act=True, blocks=[
        (0, _col(ec4_b), [
            (w4[1], 0, 0), (w4[0], 0, -1), (w4[2], 0, 1),
        ])])

    return [l1, l2, l3, l4]


def _build_dec_layers(dc2_w, dc2_b, dc3_w, dc3_b, dc4_w, dc4_b, dc5_w, dc5_b):
    dw2 = jnp.transpose(dc2_w, (2, 1, 0))
    dw3 = jnp.transpose(dc3_w, (2, 1, 0))
    dw4 = jnp.transpose(dc4_w, (2, 1, 0))
    dw5 = jnp.transpose(dc5_w, (2, 1, 0))

    l2 = dict(act=True, blocks=[
        (0, _col(dc2_b), [
            (dw2[1], 0, 0), (dw2[0], 0, 1), (dw2[2], 0, -1),
        ])])

    b3 = _col(dc3_b)
    l3 = dict(act=True, blocks=[
        (0, b3, [(dw3[1], 0, 0), (dw3[3], 0, -1)]),
        (256, b3, [(dw3[0], 0, 1), (dw3[2], 0, 0)]),
    ])

    b4 = _col(dc4_b)
    w_pair = _wblock(256, 512, [
        (0, 256, dw4[0]), (0, 0, dw4[2]),
        (128, 256, dw4[1]), (128, 0, dw4[3]),
    ])
    l4 = dict(act=True, blocks=[
        (0, b4, [(dw4[1], 0, 0), (dw4[3], 256, -1)]),
        (128, jnp.tile(b4, (2, 1)), [(w_pair, 0, 0)]),
        (384, b4, [(dw4[0], 0, 1), (dw4[2], 256, 0)]),
    ])

    sh0 = [
        (0, 0, dw5[1]),
        (2, 128, dw5[0]), (2, 0, dw5[2]),
        (4, 128, dw5[1]), (4, 0, dw5[3]),
        (6, 256, dw5[0]), (6, 128, dw5[2]),
        (8, 256, dw5[1]), (8, 128, dw5[3]),
        (10, 384, dw5[0]), (10, 256, dw5[2]),
        (12, 384, dw5[1]), (12, 256, dw5[3]),
        (14, 384, dw5[2]),
    ]
    l5 = dict(act=False, blocks=[
        (0, jnp.tile(_col(dc5_b), (8, 1)), [
            (_wblock(16, 512, sh0), 0, 0),
            (_wblock(16, 128, [(0, 0, dw5[3])]), 384, -1),
            (_wblock(16, 128, [(14, 0, dw5[0])]), 0, 1),
        ])])

    return [l2, l3, l4, l5]


def _flatten_layers(layers):
    ops = []
    spec = []
    for layer in layers:
        lb = []
        for (out0, bias, groups) in layer['blocks']:
            bi = len(ops)
            ops.append(bias)
            lg = []
            for (wv, in0, sh) in groups:
                wi = len(ops)
                ops.append(wv)
                lg.append((wi, in0, sh))
            lb.append((out0, bi, lg))
        spec.append(dict(act=layer['act'], blocks=lb))
    return ops, spec


def _bind_layer(spec_layer, vals):
    return [(out0, vals[bi], [(vals[wi], in0, sh) for (wi, in0, sh) in lg])
            for (out0, bi, lg) in spec_layer['blocks']]


def _full_spec(v):
    nd = v.ndim
    return pl.BlockSpec(v.shape, lambda i, _n=nd: (0,) * _n)


def _params():
    return pltpu.CompilerParams(
        dimension_semantics=("parallel",),
        vmem_limit_bytes=60 * 1024 * 1024,
    )


def _scratch(rows):
    return pltpu.VMEM((rows, _TLAT + 2 * _PAD), _F32)


def _vq_body(flat_ref, embt_ref, emb_ref, embsq_ref, q_ref, loss_ref):
    emb = emb_ref[...]
    embt = embt_ref[...]
    emb_sq = embsq_ref[...]
    n = flat_ref.shape[0]
    rt = 256

    def tile(j, ss):
        ft = flat_ref[pl.ds(j * rt, rt), :]
        scores = emb_sq - 2.0 * _dot(ft, embt)
        iota = jax.lax.broadcasted_iota(jnp.int32, scores.shape, 1)
        m = jnp.min(scores, axis=1, keepdims=True)
        idx = jnp.min(jnp.where(scores == m, iota, scores.shape[1]),
                      axis=1, keepdims=True)
        onehot = (iota == idx).astype(_F32)
        qt = _dot(onehot, emb)
        q_ref[pl.ds(j * rt, rt), :] = qt
        diff = qt - ft
        return ss + jnp.sum(diff * diff)

    ss = jax.lax.fori_loop(0, n // rt, tile, jnp.zeros((), _F32))
    loss_ref[...] = jnp.full((1, 1, 128), ss, _F32)


def kernel(x, ec1_w, ec1_b, ec2_w, ec2_b, ec3_w, ec3_b, ec4_w, ec4_b,
           ec5_w, ec5_b, qc_w, qc_b, emb, dc1_w, dc1_b, dc2_w, dc2_b,
           dc3_w, dc3_b, dc4_w, dc4_b, dc5_w, dc5_b):
    b_sz, c_in, t_sz = x.shape
    t_lat = t_sz // 8

    xph = x.reshape(b_sz, 2, t_lat, 8).transpose(0, 3, 1, 2) \
           .reshape(b_sz, 16, t_lat)

    enc_layers = _build_enc_layers(ec1_w, ec1_b, ec2_w, ec2_b,
                                   ec3_w, ec3_b, ec4_w, ec4_b)
    enc_ops, enc_spec = _flatten_layers(enc_layers)
    w5, b5 = ec5_w[:, :, 0], _col(ec5_b)
    wq, bq = qc_w[:, :, 0], _col(qc_b)
    n_enc = len(enc_ops)

    def enc_body(x_ref, *refs):
        vals = [refs[i][...] for i in range(n_enc)]
        w5v, b5v, wqv, bqv = (refs[n_enc][...], refs[n_enc + 1][...],
                              refs[n_enc + 2][...], refs[n_enc + 3][...])
        h_ref = refs[n_enc + 4]
        s = refs[n_enc + 5:]
        s[0][:, _PAD:_PAD + _TLAT] = x_ref[0]
        _zero_guards(s[0])
        for li in range(4):
            _run_layer(s[li], s[li + 1], _bind_layer(enc_spec[li], vals),
                       True, _PAD)
            _zero_guards(s[li + 1])

        def tile(j, _):
            seg = s[4][:, pl.ds(j * _TT + _PAD, _TT)]
            h5 = _lrelu(_dot(w5v, seg) + b5v)
            h_ref[0, :, pl.ds(j * _TT, _TT)] = _dot(wqv, h5) + bqv
            return 0

        jax.lax.fori_loop(0, _TLAT // _TT, tile, 0)

    enc_all = tuple(enc_ops) + (w5, b5, wq, bq)
    h = pl.pallas_call(
        enc_body,
        grid=(b_sz,),
        in_specs=[pl.BlockSpec((1, 16, t_lat), lambda i: (i, 0, 0))]
        + [_full_spec(v) for v in enc_all],
        out_specs=pl.BlockSpec((1, 64, t_lat), lambda i: (i, 0, 0)),
        out_shape=jax.ShapeDtypeStruct((b_sz, 64, t_lat), _F32),
        scratch_shapes=[_scratch(16), _scratch(512), _scratch(512),
                        _scratch(256), _scratch(256)],
        compiler_params=_params(),
    )(xph, *enc_all)

    n_rows = b_sz * 64 * t_lat // 64
    flat = h.reshape(n_rows, 64)
    rows_blk = n_rows // b_sz
    embt = emb.T
    emb_sq = jnp.sum(emb * emb, axis=1)[None, :]
    qflat, losses = pl.pallas_call(
        _vq_body,
        grid=(b_sz,),
        in_specs=[pl.BlockSpec((rows_blk, 64), lambda i: (i, 0)),
                  _full_spec(embt), _full_spec(emb), _full_spec(emb_sq)],
        out_specs=(pl.BlockSpec((rows_blk, 64), lambda i: (i, 0)),
                   pl.BlockSpec((1, 1, 128), lambda i: (i, 0, 0))),
        out_shape=(jax.ShapeDtypeStruct((n_rows, 64), _F32),
                   jax.ShapeDtypeStruct((b_sz, 1, 128), _F32)),
        compiler_params=_params(),
    )(flat, embt, emb, emb_sq)

    q = qflat.reshape(b_sz, 64, t_lat)

    dec_layers = _build_dec_layers(dc2_w, dc2_b, dc3_w, dc3_b,
                                   dc4_w, dc4_b, dc5_w, dc5_b)
    dec_ops, dec_spec = _flatten_layers(dec_layers)
    dw1, db1 = dc1_w[:, :, 0].T, _col(dc1_b)
    n_dec = len(dec_ops)

    def dec_body(q_ref, *refs):
        vals = [refs[i][...] for i in range(n_dec)]
        dw1v, db1v = refs[n_dec][...], refs[n_dec + 1][...]
        out_ref = refs[n_dec + 2]
        s = refs[n_dec + 3:]

        def tile(j, _):
            seg = q_ref[0, :, pl.ds(j * _TT, _TT)]
            s[0][:, pl.ds(j * _TT + _PAD, _TT)] = _lrelu(_dot(dw1v, seg)
                                                         + db1v)
            return 0

        jax.lax.fori_loop(0, _TLAT // _TT, tile, 0)
        _zero_guards(s[0])
        for li in range(3):
            _run_layer(s[li], s[li + 1], _bind_layer(dec_spec[li], vals),
                       True, _PAD)
            _zero_guards(s[li + 1])
        _run_layer(s[3], out_ref.at[0], _bind_layer(dec_spec[3], vals),
                   False, 0)

    dec_all = tuple(dec_ops) + (dw1, db1)
    dph = pl.pallas_call(
        dec_body,
        grid=(b_sz,),
        in_specs=[pl.BlockSpec((1, 64, t_lat), lambda i: (i, 0, 0))]
        + [_full_spec(v) for v in dec_all],
        out_specs=pl.BlockSpec((1, 16, t_lat), lambda i: (i, 0, 0)),
        out_shape=jax.ShapeDtypeStruct((b_sz, 16, t_lat), _F32),
        scratch_shapes=[_scratch(256), _scratch(256), _scratch(512),
                        _scratch(512)],
        compiler_params=_params(),
    )(q, *dec_all)

    d = dph.reshape(b_sz, 8, 2, t_lat).transpose(0, 2, 3, 1) \
           .reshape(b_sz, 2, t_sz)
    latent_loss = 1.25 * jnp.sum(losses[:, 0, 0]) / (b_sz * 64 * t_lat)
    return (d, latent_loss)


# speedup vs baseline: 1.2200x; 1.2200x over previous
import jax
import jax.numpy as jnp
from jax.experimental import pallas as pl
from jax.experimental.pallas import tpu as pltpu

_F32 = jnp.float32
_PAD = 128
_TT = 1024
_TLAT = 2048


def _lrelu(v):
    return jnp.where(v >= 0, v, 0.01 * v)


def _dot(a, b):
    return jnp.dot(a, b, preferred_element_type=_F32)


def _zero_guards(ref, t=_TLAT):
    c = ref.shape[0]
    ref[:, _PAD - 1:_PAD] = jnp.zeros((c, 1), _F32)
    ref[:, _PAD + t:_PAD + t + 1] = jnp.zeros((c, 1), _F32)


def _wblock(rows, kg, placements):
    w = jnp.zeros((rows, kg), _F32)
    for (r0, k0, m) in placements:
        w = w.at[r0:r0 + m.shape[0], k0:k0 + m.shape[1]].set(m)
    return w


def _run_layer(in_ref, out_ref, blocks, act, out_off):
    def tile(j, _):
        t0 = j * _TT
        for (out0, bias, groups) in blocks:
            acc = None
            for (wv, in0, sh) in groups:
                kg = wv.shape[1]
                sa = in_ref[in0:in0 + kg, pl.ds(t0, _TT + 2 * _PAD)]
                seg = sa[:, _PAD + sh:_PAD + sh + _TT]
                d = _dot(wv, seg)
                acc = d if acc is None else acc + d
            v = acc + bias
            if act:
                v = _lrelu(v)
            nr = v.shape[0]
            out_ref[out0:out0 + nr, pl.ds(t0 + out_off, _TT)] = v
        return 0

    jax.lax.fori_loop(0, _TLAT // _TT, tile, 0)


def _col(v):
    return v.reshape(-1, 1)


def _build_enc_layers(ec1_w, ec1_b, ec2_w, ec2_b, ec3_w, ec3_b, ec4_w, ec4_b):
    w1 = jnp.transpose(ec1_w, (2, 0, 1))
    w2 = jnp.transpose(ec2_w, (2, 0, 1))
    w3 = jnp.transpose(ec3_w, (2, 0, 1))
    w4 = jnp.transpose(ec4_w, (2, 0, 1))

    b1 = _col(ec1_b)
    blocks1 = []
    for q in range(4):
        groups = []
        for dk in range(4):
            s = 2 * q + dk - 1
            groups.append((w1[dk], (s % 8) * 2, s // 8))
        blocks1.append((q * 128, b1, groups))
    l1 = dict(act=True, blocks=blocks1)

    b2 = _col(ec2_b)
    blocks2 = []
    for q in range(2):
        groups = []
        for dk in range(4):
            s = 2 * q + dk - 1
            groups.append((w2[dk], (s % 4) * 128, s // 4))
        blocks2.append((q * 256, b2, groups))
    l2 = dict(act=True, blocks=blocks2)

    l3 = dict(act=True, blocks=[
        (0, _col(ec3_b), [
            (w3[0], 256, -1), (w3[1], 0, 0), (w3[2], 256, 0), (w3[3], 0, 1),
        ])])

    l4 = dict(act=True, blocks=[
        (0, _col(ec4_b), [
            (w4[1], 0, 0), (w4[0], 0, -1), (w4[2], 0, 1),
        ])])

    return [l1, l2, l3, l4]


def _build_dec_layers(dc2_w, dc2_b, dc3_w, dc3_b, dc4_w, dc4_b, dc5_w, dc5_b):
    dw2 = jnp.transpose(dc2_w, (2, 1, 0))
    dw3 = jnp.transpose(dc3_w, (2, 1, 0))
    dw4 = jnp.transpose(dc4_w, (2, 1, 0))
    dw5 = jnp.transpose(dc5_w, (2, 1, 0))

    l2 = dict(act=True, blocks=[
        (0, _col(dc2_b), [
            (dw2[1], 0, 0), (dw2[0], 0, 1), (dw2[2], 0, -1),
        ])])

    b3 = _col(dc3_b)
    l3 = dict(act=True, blocks=[
        (0, b3, [(dw3[1], 0, 0), (dw3[3], 0, -1)]),
        (256, b3, [(dw3[0], 0, 1), (dw3[2], 0, 0)]),
    ])

    b4 = _col(dc4_b)
    l4 = dict(act=True, blocks=[
        (0, b4, [(dw4[1], 0, 0), (dw4[3], 256, -1)]),
        (128, b4, [(dw4[0], 256, 0), (dw4[2], 0, 0)]),
        (256, b4, [(dw4[1], 256, 0), (dw4[3], 0, 0)]),
        (384, b4, [(dw4[0], 0, 1), (dw4[2], 256, 0)]),
    ])

    b5 = _col(dc5_b)
    blocks5 = []
    for a in range(4):
        blocks5.append((4 * a, b5, [
            (dw5[1], a * 128, 0),
            (dw5[3], ((a - 1) % 4) * 128, -1 if a == 0 else 0)]))
        blocks5.append((4 * a + 2, b5, [
            (dw5[0], ((a + 1) % 4) * 128, 1 if a == 3 else 0),
            (dw5[2], a * 128, 0)]))
    l5 = dict(act=False, blocks=blocks5)

    return [l2, l3, l4, l5]


def _flatten_layers(layers):
    ops = []
    spec = []
    for layer in layers:
        lb = []
        for (out0, bias, groups) in layer['blocks']:
            bi = len(ops)
            ops.append(bias)
            lg = []
            for (wv, in0, sh) in groups:
                wi = len(ops)
                ops.append(wv)
                lg.append((wi, in0, sh))
            lb.append((out0, bi, lg))
        spec.append(dict(act=layer['act'], blocks=lb))
    return ops, spec


def _bind_layer(spec_layer, vals):
    return [(out0, vals[bi], [(vals[wi], in0, sh) for (wi, in0, sh) in lg])
            for (out0, bi, lg) in spec_layer['blocks']]


def _full_spec(v):
    nd = v.ndim
    return pl.BlockSpec(v.shape, lambda i, _n=nd: (0,) * _n)


def _params():
    return pltpu.CompilerParams(
        dimension_semantics=("parallel",),
        vmem_limit_bytes=60 * 1024 * 1024,
    )


def _scratch(rows):
    return pltpu.VMEM((rows, _TLAT + 2 * _PAD), _F32)


def _vq_body(flat_ref, embt_ref, emb_ref, embsq_ref, q_ref, loss_ref):
    emb = emb_ref[...]
    embt = embt_ref[...]
    emb_sq = embsq_ref[...]
    n = flat_ref.shape[0]
    rt = 256

    def tile(j, ss):
        ft = flat_ref[pl.ds(j * rt, rt), :]
        scores = emb_sq - 2.0 * _dot(ft, embt)
        iota = jax.lax.broadcasted_iota(jnp.int32, scores.shape, 1)
        m = jnp.min(scores, axis=1, keepdims=True)
        idx = jnp.min(jnp.where(scores == m, iota, scores.shape[1]),
                      axis=1, keepdims=True)
        onehot = (iota == idx).astype(_F32)
        qt = _dot(onehot, emb)
        q_ref[pl.ds(j * rt, rt), :] = qt
        diff = qt - ft
        return ss + jnp.sum(diff * diff)

    ss = jax.lax.fori_loop(0, n // rt, tile, jnp.zeros((), _F32))
    loss_ref[...] = jnp.full((1, 1, 128), ss, _F32)


def kernel(x, ec1_w, ec1_b, ec2_w, ec2_b, ec3_w, ec3_b, ec4_w, ec4_b,
           ec5_w, ec5_b, qc_w, qc_b, emb, dc1_w, dc1_b, dc2_w, dc2_b,
           dc3_w, dc3_b, dc4_w, dc4_b, dc5_w, dc5_b):
    b_sz, c_in, t_sz = x.shape
    t_lat = t_sz // 8

    xph = x.reshape(b_sz, 2, t_lat, 8).transpose(0, 3, 1, 2) \
           .reshape(b_sz, 16, t_lat)

    enc_layers = _build_enc_layers(ec1_w, ec1_b, ec2_w, ec2_b,
                                   ec3_w, ec3_b, ec4_w, ec4_b)
    enc_ops, enc_spec = _flatten_layers(enc_layers)
    w5, b5 = ec5_w[:, :, 0], _col(ec5_b)
    wq, bq = qc_w[:, :, 0], _col(qc_b)
    n_enc = len(enc_ops)

    def enc_body(x_ref, *refs):
        vals = [refs[i][...] for i in range(n_enc)]
        w5v, b5v, wqv, bqv = (refs[n_enc][...], refs[n_enc + 1][...],
                              refs[n_enc + 2][...], refs[n_enc + 3][...])
        h_ref = refs[n_enc + 4]
        s = refs[n_enc + 5:]
        s[0][:, _PAD:_PAD + _TLAT] = x_ref[0]
        _zero_guards(s[0])
        for li in range(4):
            _run_layer(s[li], s[li + 1], _bind_layer(enc_spec[li], vals),
                       True, _PAD)
            _zero_guards(s[li + 1])

        def tile(j, _):
            seg = s[4][:, pl.ds(j * _TT + _PAD, _TT)]
            h5 = _lrelu(_dot(w5v, seg) + b5v)
            h_ref[0, :, pl.ds(j * _TT, _TT)] = _dot(wqv, h5) + bqv
            return 0

        jax.lax.fori_loop(0, _TLAT // _TT, tile, 0)

    enc_all = tuple(enc_ops) + (w5, b5, wq, bq)
    h = pl.pallas_call(
        enc_body,
        grid=(b_sz,),
        in_specs=[pl.BlockSpec((1, 16, t_lat), lambda i: (i, 0, 0))]
        + [_full_spec(v) for v in enc_all],
        out_specs=pl.BlockSpec((1, 64, t_lat), lambda i: (i, 0, 0)),
        out_shape=jax.ShapeDtypeStruct((b_sz, 64, t_lat), _F32),
        scratch_shapes=[_scratch(16), _scratch(512), _scratch(512),
                        _scratch(256), _scratch(256)],
        compiler_params=_params(),
    )(xph, *enc_all)

    n_rows = b_sz * 64 * t_lat // 64
    flat = h.reshape(n_rows, 64)
    rows_blk = n_rows // b_sz
    embt = emb.T
    emb_sq = jnp.sum(emb * emb, axis=1)[None, :]
    qflat, losses = pl.pallas_call(
        _vq_body,
        grid=(b_sz,),
        in_specs=[pl.BlockSpec((rows_blk, 64), lambda i: (i, 0)),
                  _full_spec(embt), _full_spec(emb), _full_spec(emb_sq)],
        out_specs=(pl.BlockSpec((rows_blk, 64), lambda i: (i, 0)),
                   pl.BlockSpec((1, 1, 128), lambda i: (i, 0, 0))),
        out_shape=(jax.ShapeDtypeStruct((n_rows, 64), _F32),
                   jax.ShapeDtypeStruct((b_sz, 1, 128), _F32)),
        compiler_params=_params(),
    )(flat, embt, emb, emb_sq)

    q = qflat.reshape(b_sz, 64, t_lat)

    dec_layers = _build_dec_layers(dc2_w, dc2_b, dc3_w, dc3_b,
                                   dc4_w, dc4_b, dc5_w, dc5_b)
    dec_ops, dec_spec = _flatten_layers(dec_layers)
    dw1, db1 = dc1_w[:, :, 0].T, _col(dc1_b)
    n_dec = len(dec_ops)

    def dec_body(q_ref, *refs):
        vals = [refs[i][...] for i in range(n_dec)]
        dw1v, db1v = refs[n_dec][...], refs[n_dec + 1][...]
        out_ref = refs[n_dec + 2]
        s = refs[n_dec + 3:]

        def tile(j, _):
            seg = q_ref[0, :, pl.ds(j * _TT, _TT)]
            s[0][:, pl.ds(j * _TT + _PAD, _TT)] = _lrelu(_dot(dw1v, seg)
                                                         + db1v)
            return 0

        jax.lax.fori_loop(0, _TLAT // _TT, tile, 0)
        _zero_guards(s[0])
        for li in range(3):
            _run_layer(s[li], s[li + 1], _bind_layer(dec_spec[li], vals),
                       True, _PAD)
            _zero_guards(s[li + 1])
        _run_layer(s[3], out_ref.at[0], _bind_layer(dec_spec[3], vals),
                   False, 0)

    dec_all = tuple(dec_ops) + (dw1, db1)
    dph = pl.pallas_call(
        dec_body,
        grid=(b_sz,),
        in_specs=[pl.BlockSpec((1, 64, t_lat), lambda i: (i, 0, 0))]
        + [_full_spec(v) for v in dec_all],
        out_specs=pl.BlockSpec((1, 16, t_lat), lambda i: (i, 0, 0)),
        out_shape=jax.ShapeDtypeStruct((b_sz, 16, t_lat), _F32),
        scratch_shapes=[_scratch(256), _scratch(256), _scratch(512),
                        _scratch(512)],
        compiler_params=_params(),
    )(q, *dec_all)

    d = dph.reshape(b_sz, 8, 2, t_lat).transpose(0, 2, 3, 1) \
           .reshape(b_sz, 2, t_sz)
    latent_loss = 1.25 * jnp.sum(losses[:, 0, 0]) / (b_sz * 64 * t_lat)
    return (d, latent_loss)


# speedup vs baseline: 1.4507x; 1.1890x over previous
import jax
import jax.numpy as jnp
from jax.experimental import pallas as pl
from jax.experimental.pallas import tpu as pltpu

_F32 = jnp.float32
_PAD = 128
_TT = 2048
_TLAT = 2048


def _lrelu(v):
    return jnp.where(v >= 0, v, 0.01 * v)


def _dot(a, b):
    return jnp.dot(a, b, preferred_element_type=_F32)


def _zero_guards(ref, t=_TLAT):
    c = ref.shape[0]
    ref[:, _PAD - 1:_PAD] = jnp.zeros((c, 1), _F32)
    ref[:, _PAD + t:_PAD + t + 1] = jnp.zeros((c, 1), _F32)


def _wblock(rows, kg, placements):
    w = jnp.zeros((rows, kg), _F32)
    for (r0, k0, m) in placements:
        w = w.at[r0:r0 + m.shape[0], k0:k0 + m.shape[1]].set(m)
    return w


def _run_layer(in_ref, out_ref, blocks, act, out_off):
    def tile(j, _):
        t0 = j * _TT
        for (out0, bias, groups) in blocks:
            acc = None
            for (wv, in0, sh) in groups:
                kg = wv.shape[1]
                sa = in_ref[in0:in0 + kg, pl.ds(t0, _TT + 2 * _PAD)]
                seg = sa[:, _PAD + sh:_PAD + sh + _TT]
                d = _dot(wv, seg)
                acc = d if acc is None else acc + d
            v = acc + bias
            if act:
                v = _lrelu(v)
            nr = v.shape[0]
            out_ref[out0:out0 + nr, pl.ds(t0 + out_off, _TT)] = v
        return 0

    jax.lax.fori_loop(0, _TLAT // _TT, tile, 0)


def _col(v):
    return v.reshape(-1, 1)


def _build_enc_layers(ec1_w, ec1_b, ec2_w, ec2_b, ec3_w, ec3_b, ec4_w, ec4_b):
    w1 = jnp.transpose(ec1_w, (2, 0, 1))
    w2 = jnp.transpose(ec2_w, (2, 0, 1))
    w3 = jnp.transpose(ec3_w, (2, 0, 1))
    w4 = jnp.transpose(ec4_w, (2, 0, 1))

    b1 = _col(ec1_b)
    blocks1 = []
    for q in range(4):
        groups = []
        for dk in range(4):
            s = 2 * q + dk - 1
            groups.append((w1[dk], (s % 8) * 2, s // 8))
        blocks1.append((q * 128, b1, groups))
    l1 = dict(act=True, blocks=blocks1)

    b2 = _col(ec2_b)
    blocks2 = []
    for q in range(2):
        groups = []
        for dk in range(4):
            s = 2 * q + dk - 1
            groups.append((w2[dk], (s % 4) * 128, s // 4))
        blocks2.append((q * 256, b2, groups))
    l2 = dict(act=True, blocks=blocks2)

    l3 = dict(act=True, blocks=[
        (0, _col(ec3_b), [
            (w3[0], 256, -1), (w3[1], 0, 0), (w3[2], 256, 0), (w3[3], 0, 1),
        ])])

    l4 = dict(act=True, blocks=[
        (0, _col(ec4_b), [
            (w4[1], 0, 0), (w4[0], 0, -1), (w4[2], 0, 1),
        ])])

    return [l1, l2, l3, l4]


def _build_dec_layers(dc2_w, dc2_b, dc3_w, dc3_b, dc4_w, dc4_b, dc5_w, dc5_b):
    dw2 = jnp.transpose(dc2_w, (2, 1, 0))
    dw3 = jnp.transpose(dc3_w, (2, 1, 0))
    dw4 = jnp.transpose(dc4_w, (2, 1, 0))
    dw5 = jnp.transpose(dc5_w, (2, 1, 0))

    l2 = dict(act=True, blocks=[
        (0, _col(dc2_b), [
            (dw2[1], 0, 0), (dw2[0], 0, 1), (dw2[2], 0, -1),
        ])])

    b3 = _col(dc3_b)
    l3 = dict(act=True, blocks=[
        (0, b3, [(dw3[1], 0, 0), (dw3[3], 0, -1)]),
        (256, b3, [(dw3[0], 0, 1), (dw3[2], 0, 0)]),
    ])

    b4 = _col(dc4_b)
    l4 = dict(act=True, blocks=[
        (0, b4, [(dw4[1], 0, 0), (dw4[3], 256, -1)]),
        (128, b4, [(dw4[0], 256, 0), (dw4[2], 0, 0)]),
        (256, b4, [(dw4[1], 256, 0), (dw4[3], 0, 0)]),
        (384, b4, [(dw4[0], 0, 1), (dw4[2], 256, 0)]),
    ])

    b5 = _col(dc5_b)
    blocks5 = []
    for a in range(4):
        blocks5.append((4 * a, b5, [
            (dw5[1], a * 128, 0),
            (dw5[3], ((a - 1) % 4) * 128, -1 if a == 0 else 0)]))
        blocks5.append((4 * a + 2, b5, [
            (dw5[0], ((a + 1) % 4) * 128, 1 if a == 3 else 0),
            (dw5[2], a * 128, 0)]))
    l5 = dict(act=False, blocks=blocks5)

    return [l2, l3, l4, l5]


def _flatten_layers(layers):
    ops = []
    spec = []
    for layer in layers:
        lb = []
        for (out0, bias, groups) in layer['blocks']:
            bi = len(ops)
            ops.append(bias)
            lg = []
            for (wv, in0, sh) in groups:
                wi = len(ops)
                ops.append(wv)
                lg.append((wi, in0, sh))
            lb.append((out0, bi, lg))
        spec.append(dict(act=layer['act'], blocks=lb))
    return ops, spec


def _bind_layer(spec_layer, vals):
    return [(out0, vals[bi], [(vals[wi], in0, sh) for (wi, in0, sh) in lg])
            for (out0, bi, lg) in spec_layer['blocks']]


def _full_spec(v):
    nd = v.ndim
    return pl.BlockSpec(v.shape, lambda i, _n=nd: (0,) * _n)


def _params():
    return pltpu.CompilerParams(
        dimension_semantics=("parallel",),
        vmem_limit_bytes=60 * 1024 * 1024,
    )


def _scratch(rows):
    return pltpu.VMEM((rows, _TLAT + 2 * _PAD), _F32)


def _vq_body(flat_ref, embt_ref, emb_ref, embsq_ref, q_ref, loss_ref):
    emb = emb_ref[...]
    embt = embt_ref[...]
    emb_sq = embsq_ref[...]
    n = flat_ref.shape[0]
    rt = 512

    def tile(j, ss):
        ft = flat_ref[pl.ds(j * rt, rt), :]
        scores = emb_sq - 2.0 * _dot(ft, embt)
        iota = jax.lax.broadcasted_iota(jnp.int32, scores.shape, 1)
        m = jnp.min(scores, axis=1, keepdims=True)
        idx = jnp.min(jnp.where(scores == m, iota, scores.shape[1]),
                      axis=1, keepdims=True)
        onehot = (iota == idx).astype(_F32)
        qt = _dot(onehot, emb)
        q_ref[pl.ds(j * rt, rt), :] = qt
        diff = qt - ft
        return ss + jnp.sum(diff * diff)

    ss = jax.lax.fori_loop(0, n // rt, tile, jnp.zeros((), _F32))
    loss_ref[...] = jnp.full((1, 1, 128), ss, _F32)


def kernel(x, ec1_w, ec1_b, ec2_w, ec2_b, ec3_w, ec3_b, ec4_w, ec4_b,
           ec5_w, ec5_b, qc_w, qc_b, emb, dc1_w, dc1_b, dc2_w, dc2_b,
           dc3_w, dc3_b, dc4_w, dc4_b, dc5_w, dc5_b):
    b_sz, c_in, t_sz = x.shape
    t_lat = t_sz // 8

    xph = x.reshape(b_sz, 2, t_lat, 8).transpose(0, 3, 1, 2) \
           .reshape(b_sz, 16, t_lat)

    enc_layers = _build_enc_layers(ec1_w, ec1_b, ec2_w, ec2_b,
                                   ec3_w, ec3_b, ec4_w, ec4_b)
    enc_ops, enc_spec = _flatten_layers(enc_layers)
    w5, b5 = ec5_w[:, :, 0], _col(ec5_b)
    wq, bq = qc_w[:, :, 0], _col(qc_b)
    n_enc = len(enc_ops)

    def enc_body(x_ref, *refs):
        vals = [refs[i][...] for i in range(n_enc)]
        w5v, b5v, wqv, bqv = (refs[n_enc][...], refs[n_enc + 1][...],
                              refs[n_enc + 2][...], refs[n_enc + 3][...])
        h_ref = refs[n_enc + 4]
        s = refs[n_enc + 5:]
        s[0][:, _PAD:_PAD + _TLAT] = x_ref[0]
        _zero_guards(s[0])
        for li in range(4):
            _run_layer(s[li], s[li + 1], _bind_layer(enc_spec[li], vals),
                       True, _PAD)
            _zero_guards(s[li + 1])

        def tile(j, _):
            seg = s[4][:, pl.ds(j * _TT + _PAD, _TT)]
            h5 = _lrelu(_dot(w5v, seg) + b5v)
            h_ref[0, :, pl.ds(j * _TT, _TT)] = _dot(wqv, h5) + bqv
            return 0

        jax.lax.fori_loop(0, _TLAT // _TT, tile, 0)

    enc_all = tuple(enc_ops) + (w5, b5, wq, bq)
    h = pl.pallas_call(
        enc_body,
        grid=(b_sz,),
        in_specs=[pl.BlockSpec((1, 16, t_lat), lambda i: (i, 0, 0))]
        + [_full_spec(v) for v in enc_all],
        out_specs=pl.BlockSpec((1, 64, t_lat), lambda i: (i, 0, 0)),
        out_shape=jax.ShapeDtypeStruct((b_sz, 64, t_lat), _F32),
        scratch_shapes=[_scratch(16), _scratch(512), _scratch(512),
                        _scratch(256), _scratch(256)],
        compiler_params=_params(),
    )(xph, *enc_all)

    n_rows = b_sz * 64 * t_lat // 64
    flat = h.reshape(n_rows, 64)
    rows_blk = n_rows // b_sz
    embt = emb.T
    emb_sq = jnp.sum(emb * emb, axis=1)[None, :]
    qflat, losses = pl.pallas_call(
        _vq_body,
        grid=(b_sz,),
        in_specs=[pl.BlockSpec((rows_blk, 64), lambda i: (i, 0)),
                  _full_spec(embt), _full_spec(emb), _full_spec(emb_sq)],
        out_specs=(pl.BlockSpec((rows_blk, 64), lambda i: (i, 0)),
                   pl.BlockSpec((1, 1, 128), lambda i: (i, 0, 0))),
        out_shape=(jax.ShapeDtypeStruct((n_rows, 64), _F32),
                   jax.ShapeDtypeStruct((b_sz, 1, 128), _F32)),
        compiler_params=_params(),
    )(flat, embt, emb, emb_sq)

    q = qflat.reshape(b_sz, 64, t_lat)

    dec_layers = _build_dec_layers(dc2_w, dc2_b, dc3_w, dc3_b,
                                   dc4_w, dc4_b, dc5_w, dc5_b)
    dec_ops, dec_spec = _flatten_layers(dec_layers)
    dw1, db1 = dc1_w[:, :, 0].T, _col(dc1_b)
    n_dec = len(dec_ops)

    def dec_body(q_ref, *refs):
        vals = [refs[i][...] for i in range(n_dec)]
        dw1v, db1v = refs[n_dec][...], refs[n_dec + 1][...]
        out_ref = refs[n_dec + 2]
        s = refs[n_dec + 3:]

        def tile(j, _):
            seg = q_ref[0, :, pl.ds(j * _TT, _TT)]
            s[0][:, pl.ds(j * _TT + _PAD, _TT)] = _lrelu(_dot(dw1v, seg)
                                                         + db1v)
            return 0

        jax.lax.fori_loop(0, _TLAT // _TT, tile, 0)
        _zero_guards(s[0])
        for li in range(3):
            _run_layer(s[li], s[li + 1], _bind_layer(dec_spec[li], vals),
                       True, _PAD)
            _zero_guards(s[li + 1])
        _run_layer(s[3], out_ref.at[0], _bind_layer(dec_spec[3], vals),
                   False, 0)

    dec_all = tuple(dec_ops) + (dw1, db1)
    dph = pl.pallas_call(
        dec_body,
        grid=(b_sz,),
        in_specs=[pl.BlockSpec((1, 64, t_lat), lambda i: (i, 0, 0))]
        + [_full_spec(v) for v in dec_all],
        out_specs=pl.BlockSpec((1, 16, t_lat), lambda i: (i, 0, 0)),
        out_shape=jax.ShapeDtypeStruct((b_sz, 16, t_lat), _F32),
        scratch_shapes=[_scratch(256), _scratch(256), _scratch(512),
                        _scratch(512)],
        compiler_params=_params(),
    )(q, *dec_all)

    d = dph.reshape(b_sz, 8, 2, t_lat).transpose(0, 2, 3, 1) \
           .reshape(b_sz, 2, t_sz)
    latent_loss = 1.25 * jnp.sum(losses[:, 0, 0]) / (b_sz * 64 * t_lat)
    return (d, latent_loss)


# speedup vs baseline: 1.5277x; 1.0531x over previous
import jax
import jax.numpy as jnp
from jax.experimental import pallas as pl
from jax.experimental.pallas import tpu as pltpu

_F32 = jnp.float32
_PAD = 128
_TT = 2048
_TLAT = 2048


def _lrelu(v):
    return jnp.where(v >= 0, v, 0.01 * v)


def _dot(a, b):
    return jnp.dot(a, b, preferred_element_type=_F32)


def _zero_guards(ref, t=_TLAT):
    c = ref.shape[0]
    ref[:, _PAD - 1:_PAD] = jnp.zeros((c, 1), _F32)
    ref[:, _PAD + t:_PAD + t + 1] = jnp.zeros((c, 1), _F32)


def _wblock(rows, kg, placements):
    w = jnp.zeros((rows, kg), _F32)
    for (r0, k0, m) in placements:
        w = w.at[r0:r0 + m.shape[0], k0:k0 + m.shape[1]].set(m)
    return w


def _run_layer(in_ref, out_ref, blocks, act, out_off):
    def tile(j, _):
        t0 = j * _TT
        for (out0, bias, groups) in blocks:
            acc = None
            for (wv, in0, sh) in groups:
                kg = wv.shape[1]
                sa = in_ref[in0:in0 + kg, pl.ds(t0, _TT + 2 * _PAD)]
                seg = sa[:, _PAD + sh:_PAD + sh + _TT]
                d = _dot(wv, seg)
                acc = d if acc is None else acc + d
            v = acc + bias
            if act:
                v = _lrelu(v)
            nr = v.shape[0]
            out_ref[out0:out0 + nr, pl.ds(t0 + out_off, _TT)] = v
        return 0

    jax.lax.fori_loop(0, _TLAT // _TT, tile, 0)


def _col(v):
    return v.reshape(-1, 1)


def _build_enc_layers(ec1_w, ec1_b, ec2_w, ec2_b, ec3_w, ec3_b, ec4_w, ec4_b):
    w1 = jnp.transpose(ec1_w, (2, 0, 1))
    w2 = jnp.transpose(ec2_w, (2, 0, 1))
    w3 = jnp.transpose(ec3_w, (2, 0, 1))
    w4 = jnp.transpose(ec4_w, (2, 0, 1))

    b1 = _col(ec1_b)
    l1 = dict(act=True, blocks=[
        (0, b1, [(jnp.concatenate([w1[1], w1[2], w1[3]], axis=1), 0, 0),
                 (w1[0], 14, -1)]),
        (128, b1, [(jnp.concatenate(list(w1), axis=1), 2, 0)]),
        (256, b1, [(jnp.concatenate(list(w1), axis=1), 6, 0)]),
        (384, b1, [(jnp.concatenate([w1[0], w1[1], w1[2]], axis=1), 10, 0),
                   (w1[3], 0, 1)]),
    ])

    b2 = _col(ec2_b)
    l2 = dict(act=True, blocks=[
        (0, b2, [(jnp.concatenate([w2[1], w2[2], w2[3]], axis=1), 0, 0),
                 (w2[0], 384, -1)]),
        (256, b2, [(jnp.concatenate([w2[0], w2[1], w2[2]], axis=1), 128, 0),
                   (w2[3], 0, 1)]),
    ])

    l3 = dict(act=True, blocks=[
        (0, _col(ec3_b), [
            (jnp.concatenate([w3[1], w3[2]], axis=1), 0, 0),
            (w3[0], 256, -1), (w3[3], 0, 1),
        ])])

    l4 = dict(act=True, blocks=[
        (0, _col(ec4_b), [
            (w4[1], 0, 0), (w4[0], 0, -1), (w4[2], 0, 1),
        ])])

    return [l1, l2, l3, l4]


def _build_dec_layers(dc2_w, dc2_b, dc3_w, dc3_b, dc4_w, dc4_b, dc5_w, dc5_b):
    dw2 = jnp.transpose(dc2_w, (2, 1, 0))
    dw3 = jnp.transpose(dc3_w, (2, 1, 0))
    dw4 = jnp.transpose(dc4_w, (2, 1, 0))
    dw5 = jnp.transpose(dc5_w, (2, 1, 0))

    l2 = dict(act=True, blocks=[
        (0, _col(dc2_b), [
            (dw2[1], 0, 0), (dw2[0], 0, 1), (dw2[2], 0, -1),
        ])])

    b3 = _col(dc3_b)
    l3 = dict(act=True, blocks=[
        (0, b3, [(dw3[1], 0, 0), (dw3[3], 0, -1)]),
        (256, b3, [(dw3[0], 0, 1), (dw3[2], 0, 0)]),
    ])

    b4 = _col(dc4_b)
    l4 = dict(act=True, blocks=[
        (0, b4, [(dw4[1], 0, 0), (dw4[3], 256, -1)]),
        (128, b4, [(jnp.concatenate([dw4[2], dw4[0]], axis=1), 0, 0)]),
        (256, b4, [(jnp.concatenate([dw4[3], dw4[1]], axis=1), 0, 0)]),
        (384, b4, [(dw4[0], 0, 1), (dw4[2], 256, 0)]),
    ])

    b5 = _col(dc5_b)
    blocks5 = []
    for a in range(4):
        blocks5.append((4 * a, b5, [
            (dw5[1], a * 128, 0),
            (dw5[3], ((a - 1) % 4) * 128, -1 if a == 0 else 0)]))
        blocks5.append((4 * a + 2, b5, [
            (dw5[0], ((a + 1) % 4) * 128, 1 if a == 3 else 0),
            (dw5[2], a * 128, 0)]))
    l5 = dict(act=False, blocks=blocks5)

    return [l2, l3, l4, l5]


def _flatten_layers(layers):
    ops = []
    spec = []
    for layer in layers:
        lb = []
        for (out0, bias, groups) in layer['blocks']:
            bi = len(ops)
            ops.append(bias)
            lg = []
            for (wv, in0, sh) in groups:
                wi = len(ops)
                ops.append(wv)
                lg.append((wi, in0, sh))
            lb.append((out0, bi, lg))
        spec.append(dict(act=layer['act'], blocks=lb))
    return ops, spec


def _bind_layer(spec_layer, vals):
    return [(out0, vals[bi], [(vals[wi], in0, sh) for (wi, in0, sh) in lg])
            for (out0, bi, lg) in spec_layer['blocks']]


def _full_spec(v):
    nd = v.ndim
    return pl.BlockSpec(v.shape, lambda i, _n=nd: (0,) * _n)


def _params():
    return pltpu.CompilerParams(
        dimension_semantics=("parallel",),
        vmem_limit_bytes=60 * 1024 * 1024,
    )


def _scratch(rows):
    return pltpu.VMEM((rows, _TLAT + 2 * _PAD), _F32)


def _vq_body(flat_ref, embt_ref, emb_ref, embsq_ref, q_ref, loss_ref):
    emb = emb_ref[...]
    embt = embt_ref[...]
    emb_sq = embsq_ref[...]
    n = flat_ref.shape[0]
    rt = 512

    def tile(j, ss):
        ft = flat_ref[pl.ds(j * rt, rt), :]
        scores = emb_sq - 2.0 * _dot(ft, embt)
        iota = jax.lax.broadcasted_iota(jnp.int32, scores.shape, 1)
        m = jnp.min(scores, axis=1, keepdims=True)
        idx = jnp.min(jnp.where(scores == m, iota, scores.shape[1]),
                      axis=1, keepdims=True)
        onehot = (iota == idx).astype(_F32)
        qt = _dot(onehot, emb)
        q_ref[pl.ds(j * rt, rt), :] = qt
        diff = qt - ft
        return ss + jnp.sum(diff * diff)

    ss = jax.lax.fori_loop(0, n // rt, tile, jnp.zeros((), _F32))
    loss_ref[...] = jnp.full((1, 1, 128), ss, _F32)


def kernel(x, ec1_w, ec1_b, ec2_w, ec2_b, ec3_w, ec3_b, ec4_w, ec4_b,
           ec5_w, ec5_b, qc_w, qc_b, emb, dc1_w, dc1_b, dc2_w, dc2_b,
           dc3_w, dc3_b, dc4_w, dc4_b, dc5_w, dc5_b):
    b_sz, c_in, t_sz = x.shape
    t_lat = t_sz // 8

    xph = x.reshape(b_sz, 2, t_lat, 8).transpose(0, 3, 1, 2) \
           .reshape(b_sz, 16, t_lat)

    enc_layers = _build_enc_layers(ec1_w, ec1_b, ec2_w, ec2_b,
                                   ec3_w, ec3_b, ec4_w, ec4_b)
    enc_ops, enc_spec = _flatten_layers(enc_layers)
    w5, b5 = ec5_w[:, :, 0], _col(ec5_b)
    wq, bq = qc_w[:, :, 0], _col(qc_b)
    n_enc = len(enc_ops)

    def enc_body(x_ref, *refs):
        vals = [refs[i][...] for i in range(n_enc)]
        w5v, b5v, wqv, bqv = (refs[n_enc][...], refs[n_enc + 1][...],
                              refs[n_enc + 2][...], refs[n_enc + 3][...])
        h_ref = refs[n_enc + 4]
        s = refs[n_enc + 5:]
        s[0][:, _PAD:_PAD + _TLAT] = x_ref[0]
        _zero_guards(s[0])
        for li in range(4):
            _run_layer(s[li], s[li + 1], _bind_layer(enc_spec[li], vals),
                       True, _PAD)
            _zero_guards(s[li + 1])

        def tile(j, _):
            seg = s[4][:, pl.ds(j * _TT + _PAD, _TT)]
            h5 = _lrelu(_dot(w5v, seg) + b5v)
            h_ref[0, :, pl.ds(j * _TT, _TT)] = _dot(wqv, h5) + bqv
            return 0

        jax.lax.fori_loop(0, _TLAT // _TT, tile, 0)

    enc_all = tuple(enc_ops) + (w5, b5, wq, bq)
    h = pl.pallas_call(
        enc_body,
        grid=(b_sz,),
        in_specs=[pl.BlockSpec((1, 16, t_lat), lambda i: (i, 0, 0))]
        + [_full_spec(v) for v in enc_all],
        out_specs=pl.BlockSpec((1, 64, t_lat), lambda i: (i, 0, 0)),
        out_shape=jax.ShapeDtypeStruct((b_sz, 64, t_lat), _F32),
        scratch_shapes=[_scratch(16), _scratch(512), _scratch(512),
                        _scratch(256), _scratch(256)],
        compiler_params=_params(),
    )(xph, *enc_all)

    n_rows = b_sz * 64 * t_lat // 64
    flat = h.reshape(n_rows, 64)
    rows_blk = n_rows // b_sz
    embt = emb.T
    emb_sq = jnp.sum(emb * emb, axis=1)[None, :]
    qflat, losses = pl.pallas_call(
        _vq_body,
        grid=(b_sz,),
        in_specs=[pl.BlockSpec((rows_blk, 64), lambda i: (i, 0)),
                  _full_spec(embt), _full_spec(emb), _full_spec(emb_sq)],
        out_specs=(pl.BlockSpec((rows_blk, 64), lambda i: (i, 0)),
                   pl.BlockSpec((1, 1, 128), lambda i: (i, 0, 0))),
        out_shape=(jax.ShapeDtypeStruct((n_rows, 64), _F32),
                   jax.ShapeDtypeStruct((b_sz, 1, 128), _F32)),
        compiler_params=_params(),
    )(flat, embt, emb, emb_sq)

    q = qflat.reshape(b_sz, 64, t_lat)

    dec_layers = _build_dec_layers(dc2_w, dc2_b, dc3_w, dc3_b,
                                   dc4_w, dc4_b, dc5_w, dc5_b)
    dec_ops, dec_spec = _flatten_layers(dec_layers)
    dw1, db1 = dc1_w[:, :, 0].T, _col(dc1_b)
    n_dec = len(dec_ops)

    def dec_body(q_ref, *refs):
        vals = [refs[i][...] for i in range(n_dec)]
        dw1v, db1v = refs[n_dec][...], refs[n_dec + 1][...]
        out_ref = refs[n_dec + 2]
        s = refs[n_dec + 3:]

        def tile(j, _):
            seg = q_ref[0, :, pl.ds(j * _TT, _TT)]
            s[0][:, pl.ds(j * _TT + _PAD, _TT)] = _lrelu(_dot(dw1v, seg)
                                                         + db1v)
            return 0

        jax.lax.fori_loop(0, _TLAT // _TT, tile, 0)
        _zero_guards(s[0])
        for li in range(3):
            _run_layer(s[li], s[li + 1], _bind_layer(dec_spec[li], vals),
                       True, _PAD)
            _zero_guards(s[li + 1])
        _run_layer(s[3], out_ref.at[0], _bind_layer(dec_spec[3], vals),
                   False, 0)

    dec_all = tuple(dec_ops) + (dw1, db1)
    dph = pl.pallas_call(
        dec_body,
        grid=(b_sz,),
        in_specs=[pl.BlockSpec((1, 64, t_lat), lambda i: (i, 0, 0))]
        + [_full_spec(v) for v in dec_all],
        out_specs=pl.BlockSpec((1, 16, t_lat), lambda i: (i, 0, 0)),
        out_shape=jax.ShapeDtypeStruct((b_sz, 16, t_lat), _F32),
        scratch_shapes=[_scratch(256), _scratch(256), _scratch(512),
                        _scratch(512)],
        compiler_params=_params(),
    )(q, *dec_all)

    d = dph.reshape(b_sz, 8, 2, t_lat).transpose(0, 2, 3, 1) \
           .reshape(b_sz, 2, t_sz)
    latent_loss = 1.25 * jnp.sum(losses[:, 0, 0]) / (b_sz * 64 * t_lat)
    return (d, latent_loss)
